# Initial kernel scaffold; baseline (speedup 1.0000x reference)
import jax
import jax.numpy as jnp
from jax.experimental import pallas as pl


def kernel(x, Wx1, bx1, Wx2, bx2, Wx3, bx3, Wx4, bx4, Wy1, by1, Wy2, by2, Wy3, by3, Wy4, by4):
    raise NotImplementedError("write your pallas kernel here")


# speedup vs baseline: 21.8127x; 21.8127x over previous
import functools

import jax
import jax.numpy as jnp
from jax import lax
from jax.experimental import pallas as pl
from jax.experimental.pallas import tpu as pltpu
from jax.experimental.pallas import tpu_sc as plsc

NCLS = 3
NTILE = 4
TSIZE = 104.0
CONF_T = 0.3
NMS_T = 0.2
NB = 16
NBOX = 20000
NPAD = 20480
CH = 1024
NCHUNK = NPAD // CH
CAP = 20032
DEAD = -1.0
NTASK = NB * NCLS
NWORK = 32


def _prep_body(x_ref, o_ref):
    cx = x_ref[0, 0:1, :]
    cy = x_ref[0, 1:2, :]
    wd = x_ref[0, 2:3, :]
    ht = x_ref[0, 3:4, :]
    cf = x_ref[0, 4:5, :]
    c0 = x_ref[0, 5:6, :]
    c1 = x_ref[0, 6:7, :]
    c2 = x_ref[0, 7:8, :]
    o_ref[0, 0:1, :] = cx - wd * 0.5
    o_ref[0, 1:2, :] = cy - ht * 0.5
    o_ref[0, 2:3, :] = cx + wd * 0.5
    o_ref[0, 3:4, :] = cy + ht * 0.5
    o_ref[0, 4:5, :] = cf
    cmax = jnp.maximum(jnp.maximum(c0, c1), c2)
    o_ref[0, 5:6, :] = cf * cmax
    cpred = jnp.where((c0 >= c1) & (c0 >= c2), 0.0, jnp.where(c1 >= c2, 1.0, 2.0))
    o_ref[0, 6:7, :] = cpred


def _prep(xin):
    return pl.pallas_call(
        _prep_body,
        grid=(NB,),
        in_specs=[pl.BlockSpec((1, 8, NPAD), lambda i: (i, 0, 0))],
        out_specs=pl.BlockSpec((1, 7, NPAD), lambda i: (i, 0, 0)),
        out_shape=jax.ShapeDtypeStruct((NB, 7, NPAD), jnp.float32),
    )(xin)


def _sc_task(t, prep_hbm, out_hbm, stage, bx1, by1, bx2, by2, bcf, bsc, feat, sem):
    img = t // NCLS
    cls_f = (t % NCLS).astype(jnp.float32)
    iota = lax.iota(jnp.int32, 16)

    def chunk_body(k, off):
        pltpu.async_copy(
            prep_hbm.at[img, :, pl.ds(k * CH, CH)], stage, sem
        ).wait()

        def vec_body(j, off2):
            b = j * 16
            cf = stage[4, pl.ds(b, 16)]
            cl = stage[6, pl.ds(b, 16)]
            m = (cf >= CONF_T) & (cl == cls_f)
            kcum = plsc.cumsum(m.astype(jnp.int32))
            pos = off2 + kcum - 1
            plsc.store_scatter(bx1, [pos], stage[0, pl.ds(b, 16)], mask=m)
            plsc.store_scatter(by1, [pos], stage[1, pl.ds(b, 16)], mask=m)
            plsc.store_scatter(bx2, [pos], stage[2, pl.ds(b, 16)], mask=m)
            plsc.store_scatter(by2, [pos], stage[3, pl.ds(b, 16)], mask=m)
            plsc.store_scatter(bcf, [pos], cf, mask=m)
            plsc.store_scatter(bsc, [pos], stage[5, pl.ds(b, 16)], mask=m)
            return off2 + jnp.max(kcum)

        return lax.fori_loop(0, CH // 16, vec_body, off)

    n0 = lax.fori_loop(0, NCHUNK, chunk_body, jnp.int32(0))
    bsc[pl.ds(n0, 16)] = jnp.full((16,), DEAD, jnp.float32)

    def am_body(j, c):
        vmax, vidx = c
        sc = bsc[pl.ds(j * 16, 16)]
        upd = sc > vmax
        return (jnp.where(upd, sc, vmax), jnp.where(upd, j * 16 + iota, vidx))

    vmax, vidx = lax.fori_loop(
        0, (n0 + 15) // 16, am_body,
        (jnp.full((16,), DEAD, jnp.float32), jnp.zeros((16,), jnp.int32)),
    )
    m0 = jnp.max(vmax)
    i0 = jnp.min(jnp.where(vmax == m0, vidx, CAP))

    zeros16 = jnp.zeros((16,), jnp.float32)

    def cond(st):
        return st[1] >= 0.0

    def body(st):
        n, _m, idx, xi, yi = st
        iv = jnp.full((16,), idx, jnp.int32)
        x1i = plsc.load_gather(bx1, [iv])
        y1i = plsc.load_gather(by1, [iv])
        x2i = plsc.load_gather(bx2, [iv])
        y2i = plsc.load_gather(by2, [iv])
        cfi = plsc.load_gather(bcf, [iv])
        a1 = (x2i - x1i + 1.0) * (y2i - y1i + 1.0)

        def sweep(j, c):
            woff, ws, wx, wy1, wy2, nvm, nvi = c
            b = j * 16
            x1 = bx1[pl.ds(b, 16)]
            y1 = by1[pl.ds(b, 16)]
            x2 = bx2[pl.ds(b, 16)]
            y2 = by2[pl.ds(b, 16)]
            cf = bcf[pl.ds(b, 16)]
            sc = bsc[pl.ds(b, 16)]
            alive = sc >= 0.0
            dw = jnp.maximum(jnp.minimum(x2i, x2) - jnp.maximum(x1i, x1) + 1.0, 0.0)
            dh = jnp.maximum(jnp.minimum(y2i, y2) - jnp.maximum(y1i, y1) + 1.0, 0.0)
            inter = dw * dh
            a2 = (x2 - x1 + 1.0) * (y2 - y1 + 1.0)
            iou = inter / (a1 + a2 - inter + 1e-16)
            supp = (iou > NMS_T) & alive
            w = jnp.where(supp, cf, 0.0)
            keep = alive & (~supp)
            kcum = plsc.cumsum(keep.astype(jnp.int32))
            pos = woff + kcum - 1
            plsc.store_scatter(bx1, [pos], x1, mask=keep)
            plsc.store_scatter(by1, [pos], y1, mask=keep)
            plsc.store_scatter(bx2, [pos], x2, mask=keep)
            plsc.store_scatter(by2, [pos], y2, mask=keep)
            plsc.store_scatter(bcf, [pos], cf, mask=keep)
            plsc.store_scatter(bsc, [pos], sc, mask=keep)
            upd = keep & (sc > nvm)
            return (
                woff + jnp.max(kcum),
                ws + w,
                wx + w * x1,
                wy1 + w * y1,
                wy2 + w * y2,
                jnp.where(upd, sc, nvm),
                jnp.where(upd, pos, nvi),
            )

        woff, ws, wx, wy1, wy2, nvm, nvi = lax.fori_loop(
            0, (n + 15) // 16, sweep,
            (jnp.int32(0), zeros16, zeros16, zeros16, zeros16,
             jnp.full((16,), DEAD, jnp.float32), jnp.zeros((16,), jnp.int32)),
        )
        bsc[pl.ds(woff, 16)] = jnp.full((16,), DEAD, jnp.float32)

        wsum_v = jnp.full((16,), jnp.sum(ws), jnp.float32)
        mx1_v = jnp.full((16,), jnp.sum(wx), jnp.float32) / wsum_v
        my1_v = jnp.full((16,), jnp.sum(wy1), jnp.float32) / wsum_v
        my2_v = jnp.full((16,), jnp.sum(wy2), jnp.float32) / wsum_v

        def tile_idx(q):
            ti = q.astype(jnp.int32)
            fl = ti - jnp.where(ti.astype(jnp.float32) > q, 1, 0)
            fl = jnp.minimum(fl, NTILE - 1)
            return ((fl % NTILE) + NTILE) % NTILE

        c_iv = jnp.full((16,), t % NCLS, jnp.int32)
        bxa = tile_idx(mx1_v * (1.0 / TSIZE)) * NCLS + c_iv
        bxb = tile_idx(my2_v * (1.0 / TSIZE)) * NCLS + c_iv
        bya = tile_idx(my1_v * (1.0 / TSIZE)) * NCLS + c_iv
        byb = tile_idx(cfi * (1.0 / TSIZE)) * NCLS + c_iv
        xi = xi + jnp.where(iota == bxa, cfi, 0.0)
        xi = xi + jnp.where((iota == bxb) & (bxa != bxb), cfi, 0.0)
        yi = yi + jnp.where(iota == bya, cfi, 0.0)
        yi = yi + jnp.where((iota == byb) & (bya != byb), cfi, 0.0)

        nm = jnp.max(nvm)
        ni = jnp.min(jnp.where(nvm == nm, nvi, CAP))
        return (woff, nm, ni, xi, yi)

    _, _, _, xi, yi = lax.while_loop(
        cond, body, (n0, m0, i0, zeros16, zeros16)
    )
    feat[0, :] = xi
    feat[1, :] = yi
    pltpu.sync_copy(feat, out_hbm.at[t])


def _sc_nms(prep):
    mesh = plsc.VectorSubcoreMesh(core_axis_name="c", subcore_axis_name="s")

    @functools.partial(
        pl.kernel,
        mesh=mesh,
        compiler_params=pltpu.CompilerParams(needs_layout_passes=False),
        out_type=jax.ShapeDtypeStruct((NTASK, 2, 16), jnp.float32),
        scratch_types=[
            pltpu.VMEM((7, CH), jnp.float32),
            pltpu.VMEM((CAP,), jnp.float32),
            pltpu.VMEM((CAP,), jnp.float32),
            pltpu.VMEM((CAP,), jnp.float32),
            pltpu.VMEM((CAP,), jnp.float32),
            pltpu.VMEM((CAP,), jnp.float32),
            pltpu.VMEM((CAP,), jnp.float32),
            pltpu.VMEM((2, 16), jnp.float32),
            pltpu.SemaphoreType.DMA,
        ],
    )
    def k(prep_hbm, out_hbm, stage, bx1, by1, bx2, by2, bcf, bsc, feat, sem):
        wid = lax.axis_index("c") * 16 + lax.axis_index("s")
        _sc_task(wid, prep_hbm, out_hbm, stage,
                 bx1, by1, bx2, by2, bcf, bsc, feat, sem)

        @pl.when(wid + NWORK < NTASK)
        def _():
            _sc_task(wid + NWORK, prep_hbm, out_hbm, stage,
                     bx1, by1, bx2, by2, bcf, bsc, feat, sem)

    return k(prep)


def _mlp_body(f_ref, wx1, bx1, wx2, bx2, wx3, bx3, wx4, bx4,
              wy1, by1, wy2, by2, wy3, by3, wy4, by4, ox_ref, oy_ref):
    br = lax.broadcasted_iota(jnp.int32, (NB, NTASK), 0)
    bc = lax.broadcasted_iota(jnp.int32, (NB, NTASK), 1)
    sel = (bc // NCLS == br).astype(jnp.float32)

    def head(fv, w1, b1, w2, b2, w3, b3, w4, b4):
        f = jax.lax.dot(sel, fv, precision=jax.lax.Precision.HIGHEST)[:, :12]
        h = jnp.maximum(jax.lax.dot(f, w1[:], precision=jax.lax.Precision.HIGHEST) + b1[:], 0.0)
        h = jnp.maximum(jax.lax.dot(h, w2[:], precision=jax.lax.Precision.HIGHEST) + b2[:], 0.0)
        h = jnp.maximum(jax.lax.dot(h, w3[:], precision=jax.lax.Precision.HIGHEST) + b3[:], 0.0)
        return jax.lax.dot(h, w4[:], precision=jax.lax.Precision.HIGHEST) + b4[:]

    ox_ref[:] = head(f_ref[:, 0, :], wx1, bx1, wx2, bx2, wx3, bx3, wx4, bx4)
    oy_ref[:] = head(f_ref[:, 1, :], wy1, by1, wy2, by2, wy3, by3, wy4, by4)


def _mlp(feats, *wb):
    return pl.pallas_call(
        _mlp_body,
        out_shape=(
            jax.ShapeDtypeStruct((NB, NTILE), jnp.float32),
            jax.ShapeDtypeStruct((NB, NTILE), jnp.float32),
        ),
    )(feats, *wb)


def kernel(x, Wx1, bx1, Wx2, bx2, Wx3, bx3, Wx4, bx4,
           Wy1, by1, Wy2, by2, Wy3, by3, Wy4, by4):
    xin = jnp.pad(jnp.transpose(x, (0, 2, 1)), ((0, 0), (0, 0), (0, NPAD - NBOX)))
    prep = _prep(xin)
    feats = _sc_nms(prep)
    out_x, out_y = _mlp(feats, Wx1, bx1, Wx2, bx2, Wx3, bx3, Wx4, bx4,
                        Wy1, by1, Wy2, by2, Wy3, by3, Wy4, by4)
    return (out_x, out_y, jnp.asarray(0.0, dtype=jnp.float32))


# speedup vs baseline: 30.8948x; 1.4164x over previous
import functools

import jax
import jax.numpy as jnp
from jax import lax
from jax.experimental import pallas as pl
from jax.experimental.pallas import tpu as pltpu
from jax.experimental.pallas import tpu_sc as plsc

NCLS = 3
NTILE = 4
TSIZE = 104.0
CONF_T = 0.3
NMS_T = 0.2
NB = 16
NBOX = 20000
NPAD = 20480
CH = 512
NCHUNK = NPAD // CH
CAP = 20032
DEAD = -1.0
NTASK = NB * NCLS
NWORK = 32


def _prep_body(x_ref, o_ref):
    cx = x_ref[0, 0:1, :]
    cy = x_ref[0, 1:2, :]
    wd = x_ref[0, 2:3, :]
    ht = x_ref[0, 3:4, :]
    cf = x_ref[0, 4:5, :]
    c0 = x_ref[0, 5:6, :]
    c1 = x_ref[0, 6:7, :]
    c2 = x_ref[0, 7:8, :]
    o_ref[0, 0:1, :] = cx - wd * 0.5
    o_ref[0, 1:2, :] = cy - ht * 0.5
    o_ref[0, 2:3, :] = cx + wd * 0.5
    o_ref[0, 3:4, :] = cy + ht * 0.5
    o_ref[0, 4:5, :] = cf
    cmax = jnp.maximum(jnp.maximum(c0, c1), c2)
    o_ref[0, 5:6, :] = cf * cmax
    cpred = jnp.where((c0 >= c1) & (c0 >= c2), 0.0, jnp.where(c1 >= c2, 1.0, 2.0))
    o_ref[0, 6:7, :] = cpred


def _prep(xin):
    return pl.pallas_call(
        _prep_body,
        grid=(NB,),
        in_specs=[pl.BlockSpec((1, 8, NPAD), lambda i: (i, 0, 0))],
        out_specs=pl.BlockSpec((1, 7, NPAD), lambda i: (i, 0, 0)),
        out_shape=jax.ShapeDtypeStruct((NB, 7, NPAD), jnp.float32),
    )(xin)


def _sc_task(t, prep_hbm, out_hbm, stage, bx1, by1, bx2, by2, bcf, bsc, feat,
             sem0, sem1):
    img = t // NCLS
    cls_f = (t % NCLS).astype(jnp.float32)
    iota = lax.iota(jnp.int32, 16)

    def proc_chunk(buf, offv):
        def vec_body(j, off2v):
            b = j * 16
            cf = stage[buf, 4, pl.ds(b, 16)]
            cl = stage[buf, 6, pl.ds(b, 16)]
            m = (cf >= CONF_T) & (cl == cls_f)
            kcum = plsc.cumsum(jnp.where(m, 1, 0))
            pos = off2v + kcum - 1
            plsc.store_scatter(bx1, [pos], stage[buf, 0, pl.ds(b, 16)], mask=m)
            plsc.store_scatter(by1, [pos], stage[buf, 1, pl.ds(b, 16)], mask=m)
            plsc.store_scatter(bx2, [pos], stage[buf, 2, pl.ds(b, 16)], mask=m)
            plsc.store_scatter(by2, [pos], stage[buf, 3, pl.ds(b, 16)], mask=m)
            plsc.store_scatter(bcf, [pos], cf, mask=m)
            plsc.store_scatter(bsc, [pos], stage[buf, 5, pl.ds(b, 16)], mask=m)
            return off2v + plsc.all_reduce_population_count(m)

        return lax.fori_loop(0, CH // 16, vec_body, offv)

    def start_chunk(k, buf, sem):
        pltpu.make_async_copy(
            prep_hbm.at[img, :, pl.ds(k * CH, CH)], stage.at[buf], sem
        ).start()

    def wait_chunk(k, buf, sem):
        pltpu.make_async_copy(
            prep_hbm.at[img, :, pl.ds(k * CH, CH)], stage.at[buf], sem
        ).wait()

    start_chunk(0, 0, sem0)

    def chunk_pair(k2, offv):
        ka = 2 * k2
        start_chunk(ka + 1, 1, sem1)
        wait_chunk(ka, 0, sem0)
        offv = proc_chunk(0, offv)

        @pl.when(ka + 2 < NCHUNK)
        def _():
            start_chunk(ka + 2, 0, sem0)

        wait_chunk(ka + 1, 1, sem1)
        return proc_chunk(1, offv)

    offv = lax.fori_loop(0, NCHUNK // 2, chunk_pair,
                         jnp.zeros((16,), jnp.int32))
    n0 = jnp.sum(offv) >> 4
    bsc[pl.ds(n0, 16)] = jnp.full((16,), DEAD, jnp.float32)

    def am_body(j, c):
        vmax, vidx = c
        sc = bsc[pl.ds(j * 16, 16)]
        upd = sc > vmax
        return (jnp.where(upd, sc, vmax), jnp.where(upd, j * 16 + iota, vidx))

    vmax, vidx = lax.fori_loop(
        0, (n0 + 15) // 16, am_body,
        (jnp.full((16,), DEAD, jnp.float32), jnp.zeros((16,), jnp.int32)),
    )
    m0 = jnp.max(vmax)
    i0 = jnp.min(jnp.where(vmax == m0, vidx, CAP))

    zeros16 = jnp.zeros((16,), jnp.float32)

    def cond(st):
        return st[1] >= 0.0

    def body(st):
        n, _m, idx, xi, yi = st
        iv = jnp.full((16,), idx, jnp.int32)
        x1i = plsc.load_gather(bx1, [iv])
        y1i = plsc.load_gather(by1, [iv])
        x2i = plsc.load_gather(bx2, [iv])
        y2i = plsc.load_gather(by2, [iv])
        cfi = plsc.load_gather(bcf, [iv])
        a1 = (x2i - x1i + 1.0) * (y2i - y1i + 1.0)

        def sweep(j, c):
            woffv, ws, wx, wy1, wy2, nvm, nvi = c
            b = j * 16
            x1 = bx1[pl.ds(b, 16)]
            y1 = by1[pl.ds(b, 16)]
            x2 = bx2[pl.ds(b, 16)]
            y2 = by2[pl.ds(b, 16)]
            cf = bcf[pl.ds(b, 16)]
            sc = bsc[pl.ds(b, 16)]
            alive = sc >= 0.0
            dw = jnp.maximum(jnp.minimum(x2i, x2) - jnp.maximum(x1i, x1) + 1.0, 0.0)
            dh = jnp.maximum(jnp.minimum(y2i, y2) - jnp.maximum(y1i, y1) + 1.0, 0.0)
            inter = dw * dh
            a2 = (x2 - x1 + 1.0) * (y2 - y1 + 1.0)
            iou = inter / (a1 + a2 - inter + 1e-16)
            supp = (iou > NMS_T) & alive
            w = jnp.where(supp, cf, 0.0)
            keep = alive & (~supp)
            kcum = plsc.cumsum(jnp.where(keep, 1, 0))
            pos = woffv + kcum - 1
            plsc.store_scatter(bx1, [pos], x1, mask=keep)
            plsc.store_scatter(by1, [pos], y1, mask=keep)
            plsc.store_scatter(bx2, [pos], x2, mask=keep)
            plsc.store_scatter(by2, [pos], y2, mask=keep)
            plsc.store_scatter(bcf, [pos], cf, mask=keep)
            plsc.store_scatter(bsc, [pos], sc, mask=keep)
            upd = keep & (sc > nvm)
            return (
                woffv + plsc.all_reduce_population_count(keep),
                ws + w,
                wx + w * x1,
                wy1 + w * y1,
                wy2 + w * y2,
                jnp.where(upd, sc, nvm),
                jnp.where(upd, pos, nvi),
            )

        woffv, ws, wx, wy1, wy2, nvm, nvi = lax.fori_loop(
            0, (n + 15) // 16, sweep,
            (jnp.zeros((16,), jnp.int32), zeros16, zeros16, zeros16, zeros16,
             jnp.full((16,), DEAD, jnp.float32), jnp.zeros((16,), jnp.int32)),
        )
        woff = jnp.sum(woffv) >> 4
        bsc[pl.ds(woff, 16)] = jnp.full((16,), DEAD, jnp.float32)

        wsum_v = jnp.full((16,), jnp.sum(ws), jnp.float32)
        mx1_v = jnp.full((16,), jnp.sum(wx), jnp.float32) / wsum_v
        my1_v = jnp.full((16,), jnp.sum(wy1), jnp.float32) / wsum_v
        my2_v = jnp.full((16,), jnp.sum(wy2), jnp.float32) / wsum_v

        def tile_idx(q):
            ti = q.astype(jnp.int32)
            fl = ti - jnp.where(ti.astype(jnp.float32) > q, 1, 0)
            fl = jnp.minimum(fl, NTILE - 1)
            return ((fl % NTILE) + NTILE) % NTILE

        c_iv = jnp.full((16,), t % NCLS, jnp.int32)
        bxa = tile_idx(mx1_v * (1.0 / TSIZE)) * NCLS + c_iv
        bxb = tile_idx(my2_v * (1.0 / TSIZE)) * NCLS + c_iv
        bya = tile_idx(my1_v * (1.0 / TSIZE)) * NCLS + c_iv
        byb = tile_idx(cfi * (1.0 / TSIZE)) * NCLS + c_iv
        xi = xi + jnp.where(iota == bxa, cfi, 0.0)
        xi = xi + jnp.where((iota == bxb) & (bxa != bxb), cfi, 0.0)
        yi = yi + jnp.where(iota == bya, cfi, 0.0)
        yi = yi + jnp.where((iota == byb) & (bya != byb), cfi, 0.0)

        nm = jnp.max(nvm)
        ni = jnp.min(jnp.where(nvm == nm, nvi, CAP))
        return (woff, nm, ni, xi, yi)

    _, _, _, xi, yi = lax.while_loop(
        cond, body, (n0, m0, i0, zeros16, zeros16)
    )
    feat[0, :] = xi
    feat[1, :] = yi
    pltpu.sync_copy(feat, out_hbm.at[t])


def _sc_nms(prep):
    mesh = plsc.VectorSubcoreMesh(core_axis_name="c", subcore_axis_name="s")

    @functools.partial(
        pl.kernel,
        mesh=mesh,
        compiler_params=pltpu.CompilerParams(needs_layout_passes=False),
        out_type=jax.ShapeDtypeStruct((NTASK, 2, 16), jnp.float32),
        scratch_types=[
            pltpu.VMEM((2, 7, CH), jnp.float32),
            pltpu.VMEM((CAP,), jnp.float32),
            pltpu.VMEM((CAP,), jnp.float32),
            pltpu.VMEM((CAP,), jnp.float32),
            pltpu.VMEM((CAP,), jnp.float32),
            pltpu.VMEM((CAP,), jnp.float32),
            pltpu.VMEM((CAP,), jnp.float32),
            pltpu.VMEM((2, 16), jnp.float32),
            pltpu.SemaphoreType.DMA,
            pltpu.SemaphoreType.DMA,
        ],
    )
    def k(prep_hbm, out_hbm, stage, bx1, by1, bx2, by2, bcf, bsc, feat,
          sem0, sem1):
        wid = lax.axis_index("s") * 2 + lax.axis_index("c")
        _sc_task(wid, prep_hbm, out_hbm, stage,
                 bx1, by1, bx2, by2, bcf, bsc, feat, sem0, sem1)

        @pl.when(wid + NWORK < NTASK)
        def _():
            _sc_task(wid + NWORK, prep_hbm, out_hbm, stage,
                     bx1, by1, bx2, by2, bcf, bsc, feat, sem0, sem1)

    return k(prep)


def _mlp_body(f_ref, wx1, bx1, wx2, bx2, wx3, bx3, wx4, bx4,
              wy1, by1, wy2, by2, wy3, by3, wy4, by4, ox_ref, oy_ref):
    br = lax.broadcasted_iota(jnp.int32, (NB, NTASK), 0)
    bc = lax.broadcasted_iota(jnp.int32, (NB, NTASK), 1)
    sel = (bc // NCLS == br).astype(jnp.float32)

    def head(fv, w1, b1, w2, b2, w3, b3, w4, b4):
        f = jax.lax.dot(sel, fv, precision=jax.lax.Precision.HIGHEST)[:, :12]
        h = jnp.maximum(jax.lax.dot(f, w1[:], precision=jax.lax.Precision.HIGHEST) + b1[:], 0.0)
        h = jnp.maximum(jax.lax.dot(h, w2[:], precision=jax.lax.Precision.HIGHEST) + b2[:], 0.0)
        h = jnp.maximum(jax.lax.dot(h, w3[:], precision=jax.lax.Precision.HIGHEST) + b3[:], 0.0)
        return jax.lax.dot(h, w4[:], precision=jax.lax.Precision.HIGHEST) + b4[:]

    ox_ref[:] = head(f_ref[:, 0, :], wx1, bx1, wx2, bx2, wx3, bx3, wx4, bx4)
    oy_ref[:] = head(f_ref[:, 1, :], wy1, by1, wy2, by2, wy3, by3, wy4, by4)


def _mlp(feats, *wb):
    return pl.pallas_call(
        _mlp_body,
        out_shape=(
            jax.ShapeDtypeStruct((NB, NTILE), jnp.float32),
            jax.ShapeDtypeStruct((NB, NTILE), jnp.float32),
        ),
    )(feats, *wb)


def kernel(x, Wx1, bx1, Wx2, bx2, Wx3, bx3, Wx4, bx4,
           Wy1, by1, Wy2, by2, Wy3, by3, Wy4, by4):
    xin = jnp.pad(jnp.transpose(x, (0, 2, 1)), ((0, 0), (0, 0), (0, NPAD - NBOX)))
    prep = _prep(xin)
    feats = _sc_nms(prep)
    out_x, out_y = _mlp(feats, Wx1, bx1, Wx2, bx2, Wx3, bx3, Wx4, bx4,
                        Wy1, by1, Wy2, by2, Wy3, by3, Wy4, by4)
    return (out_x, out_y, jnp.asarray(0.0, dtype=jnp.float32))
